# baseline (speedup 1.0000x reference)
import jax
import jax.numpy as jnp
from jax import lax
from jax.experimental import pallas as pl
from jax.experimental.pallas import tpu as pltpu
from jax.experimental.pallas import tpu_sc as plsc

N = 10000
E = 320000
H = 128
H2 = 2 * H
NCORE = 2
NSUB = 16
NW = NCORE * NSUB
SEG = 5
ESEG = E // SEG
EPW = ESEG // NW
CHUNK = 80
NCHUNK = EPW // CHUNK
NPAIR = (NCHUNK + 1) // 2
NPAD = 10240
ROWS_PER_SUB = NPAD // NSUB
ZROWS = 128

_f32 = jnp.float32
_i32 = jnp.int32


def _mm(a, b):
    return jnp.dot(a, b, preferred_element_type=_f32)


def _ln(h, g, be):
    m = jnp.mean(h, axis=-1, keepdims=True)
    d = h - m
    v = jnp.mean(d * d, axis=-1, keepdims=True)
    return d * lax.rsqrt(v + 1e-5) * g + be


def _full(shape):
    nd = len(shape)
    return pl.BlockSpec(shape, lambda i, _nd=nd: (0,) * _nd)


NBLK = 2000
EBLK = 2000
SBLKS = ESEG // EBLK


def _node0_body(x, W1, b1, W2, b2, g, be, W1pq, xo, to):
    h = jnp.maximum(_mm(x[...], W1[...]) + b1[...][None, :], 0.0)
    h = jnp.maximum(_mm(h, W2[...]) + b2[...][None, :], 0.0)
    xn = _ln(h, g[...][None, :], be[...][None, :])
    xo[...] = xn
    to[...] = _mm(xn, W1pq[...])


def _edge0_body(ea, W1, b1, W2, b2, g, be, W1c, b1e, co):
    h = jnp.maximum(_mm(ea[...], W1[...]) + b1[...][None, :], 0.0)
    h = jnp.maximum(_mm(h, W2[...]) + b2[...][None, :], 0.0)
    e0 = _ln(h, g[...][None, :], be[...][None, :])
    co[...] = _mm(e0, W1c[...]) + b1e[...][None, :]


def _edge_main_body(p1, p2, c, W2, b2, g, be, W1c, b1e, mo, co):
    cc = c[...]
    h = jnp.maximum(_mm(jnp.maximum(p1[...] + cc, 0.0), W2[...]) + b2[...][None, :], 0.0)
    mo[...] = _ln(h, g[...][None, :], be[...][None, :])
    h2 = jnp.maximum(_mm(jnp.maximum(p2[...] + cc, 0.0), W2[...]) + b2[...][None, :], 0.0)
    t = _ln(h2, g[...][None, :], be[...][None, :])
    co[...] = _mm(t, W1c[...]) + b1e[...][None, :]


def _edge_last_body(p1, c, W2, b2, g, be, mo):
    h = jnp.maximum(_mm(jnp.maximum(p1[...] + c[...], 0.0), W2[...]) + b2[...][None, :], 0.0)
    mo[...] = _ln(h, g[...][None, :], be[...][None, :])


def _node_upd_body(ag, x, Wna, Wnb, b1n, W2n, b2n, gn, ben, W1pq, xo, to):
    a = ag[...][0] + ag[...][1]
    pre = _mm(a, Wna[...]) + _mm(x[...], Wnb[...]) + b1n[...][None, :]
    h = jnp.maximum(_mm(jnp.maximum(pre, 0.0), W2n[...]) + b2n[...][None, :], 0.0)
    xn = _ln(h, gn[...][None, :], ben[...][None, :])
    xo[...] = xn
    to[...] = _mm(xn, W1pq[...])


def _node_upd_split_body(ag, x, Wna, Wnb, b1n, W2n, b2n, gn, ben, W1p, W1q,
                         xo, tpo, tqo):
    a = ag[...][0] + ag[...][1]
    pre = _mm(a, Wna[...]) + _mm(x[...], Wnb[...]) + b1n[...][None, :]
    h = jnp.maximum(_mm(jnp.maximum(pre, 0.0), W2n[...]) + b2n[...][None, :], 0.0)
    xn = _ln(h, gn[...][None, :], ben[...][None, :])
    xo[...] = xn
    tpo[...] = _mm(xn, W1p[...])
    tqo[...] = _mm(xn, W1q[...])


def _node_fin_body(ag, x, Wna, Wnb, b1n, W2n, b2n, gn, ben, Wf1, bf1, Wf2,
                   bf2, oo):
    a = ag[...][0] + ag[...][1]
    pre = _mm(a, Wna[...]) + _mm(x[...], Wnb[...]) + b1n[...][None, :]
    h = jnp.maximum(_mm(jnp.maximum(pre, 0.0), W2n[...]) + b2n[...][None, :], 0.0)
    xn = _ln(h, gn[...][None, :], ben[...][None, :])
    h2 = jnp.maximum(_mm(xn, Wf1[...]) + bf1[...][None, :], 0.0)
    oo[...] = _mm(h2, Wf2[...]) + bf2[...][None, :]


def _row_spec(blk, width):
    return pl.BlockSpec((blk, width), lambda i: (i, 0))


def _seg_spec(blk, width, seg):
    off = seg * SBLKS
    return pl.BlockSpec((blk, width), lambda i, _o=off: (_o + i, 0))


_AG_SPEC = pl.BlockSpec((2, NBLK, H), lambda i: (0, i, 0))
_AG_SHAPE = jax.ShapeDtypeStruct((NCORE, NPAD, H), _f32)


def _node0_call(x, W1, b1, W2, b2, g, be, W1pq):
    nb = N // NBLK
    wspecs = [_full(w.shape) for w in (W1, b1, W2, b2, g, be, W1pq)]
    return pl.pallas_call(
        _node0_body,
        grid=(nb,),
        in_specs=[_row_spec(NBLK, H)] + wspecs,
        out_specs=[_row_spec(NBLK, H), _row_spec(NBLK, H2)],
        out_shape=[jax.ShapeDtypeStruct((N, H), _f32),
                   jax.ShapeDtypeStruct((N, H2), _f32)],
    )(x, W1, b1, W2, b2, g, be, W1pq)


def _edge0_call(ea, seg, W1, b1, W2, b2, g, be, W1c, b1e):
    ws = (W1, b1, W2, b2, g, be, W1c, b1e)
    wspecs = [_full(w.shape) for w in ws]
    return pl.pallas_call(
        _edge0_body,
        grid=(SBLKS,),
        in_specs=[_seg_spec(EBLK, ea.shape[1], seg)] + wspecs,
        out_specs=_row_spec(EBLK, H),
        out_shape=jax.ShapeDtypeStruct((ESEG, H), _f32),
    )(ea, *ws)


def _edge_main_call(p1, p2, c, W2, b2, g, be, W1c, b1e):
    ws = (W2, b2, g, be, W1c, b1e)
    wspecs = [_full(w.shape) for w in ws]
    return pl.pallas_call(
        _edge_main_body,
        grid=(SBLKS,),
        in_specs=[_row_spec(EBLK, H)] * 3 + wspecs,
        out_specs=[_row_spec(EBLK, H)] * 2,
        out_shape=[jax.ShapeDtypeStruct((ESEG, H), _f32)] * 2,
    )(p1, p2, c, *ws)


def _edge_last_call(p1, c, W2, b2, g, be):
    ws = (W2, b2, g, be)
    wspecs = [_full(w.shape) for w in ws]
    return pl.pallas_call(
        _edge_last_body,
        grid=(SBLKS,),
        in_specs=[_row_spec(EBLK, H)] * 2 + wspecs,
        out_specs=_row_spec(EBLK, H),
        out_shape=jax.ShapeDtypeStruct((ESEG, H), _f32),
    )(p1, c, *ws)


def _node_upd_call(ag, x, Wna, Wnb, b1n, W2n, b2n, gn, ben, W1pq):
    nb = N // NBLK
    ws = (Wna, Wnb, b1n, W2n, b2n, gn, ben, W1pq)
    wspecs = [_full(w.shape) for w in ws]
    return pl.pallas_call(
        _node_upd_body,
        grid=(nb,),
        in_specs=[_AG_SPEC, _row_spec(NBLK, H)] + wspecs,
        out_specs=[_row_spec(NBLK, H), _row_spec(NBLK, H2)],
        out_shape=[jax.ShapeDtypeStruct((N, H), _f32),
                   jax.ShapeDtypeStruct((N, H2), _f32)],
    )(ag, x, *ws)


def _node_upd_split_call(ag, x, Wna, Wnb, b1n, W2n, b2n, gn, ben, W1p, W1q):
    nb = N // NBLK
    ws = (Wna, Wnb, b1n, W2n, b2n, gn, ben, W1p, W1q)
    wspecs = [_full(w.shape) for w in ws]
    return pl.pallas_call(
        _node_upd_split_body,
        grid=(nb,),
        in_specs=[_AG_SPEC, _row_spec(NBLK, H)] + wspecs,
        out_specs=[_row_spec(NBLK, H)] * 3,
        out_shape=[jax.ShapeDtypeStruct((N, H), _f32)] * 3,
    )(ag, x, *ws)


def _node_fin_call(ag, x, Wna, Wnb, b1n, W2n, b2n, gn, ben, Wf1, bf1, Wf2, bf2):
    nb = N // NBLK
    ws = (Wna, Wnb, b1n, W2n, b2n, gn, ben, Wf1, bf1, Wf2, bf2)
    wspecs = [_full(w.shape) for w in ws]
    dout = Wf2.shape[1]
    return pl.pallas_call(
        _node_fin_body,
        grid=(nb,),
        in_specs=[_AG_SPEC, _row_spec(NBLK, H)] + wspecs,
        out_specs=_row_spec(NBLK, dout),
        out_shape=jax.ShapeDtypeStruct((N, dout), _f32),
    )(ag, x, *ws)


_MESH = plsc.VectorSubcoreMesh(core_axis_name="c", subcore_axis_name="s")


def _wid():
    return lax.axis_index("s") * NCORE + lax.axis_index("c")


def _make_gather_both(seg):
    seg_base = seg * ESEG
    out_type = (jax.ShapeDtypeStruct((ESEG, H), _f32),
                jax.ShapeDtypeStruct((ESEG, H), _f32))
    scratch = (
        [pltpu.VMEM((CHUNK,), _i32)] * 4 +
        [pltpu.VMEM((CHUNK, H2), _f32)] * 4 +
        [pltpu.VMEM((CHUNK, H), _f32)] * 4 +
        [pltpu.SemaphoreType.DMA] * 6
    )

    def body(th, colh, rowh, dummy, o1h, o2h,
             colb0, colb1, rowb0, rowb1, gc0, gc1, gr0, gr1,
             o10, o20, o11, o21, si0, si1, sg0, sg1, sw0, sw1):
        del dummy
        colb = (colb0, colb1)
        rowb = (rowb0, rowb1)
        gc = (gc0, gc1)
        gr = (gr0, gr1)
        ob = ((o10, o20), (o11, o21))
        si = (si0, si1)
        sg = (sg0, sg1)
        sw = (sw0, sw1)
        outs = (o1h, o2h)
        obase0 = _wid() * EPW
        base0 = seg_base + obase0

        def idx_start(g, b):
            base = base0 + g * CHUNK
            pltpu.async_copy(colh.at[pl.ds(base, CHUNK)], colb[b], si[b])
            pltpu.async_copy(rowh.at[pl.ds(base, CHUNK)], rowb[b], si[b])

        def idx_wait(b):
            pltpu.make_async_copy(colh.at[pl.ds(0, CHUNK)], colb[b], si[b]).wait()
            pltpu.make_async_copy(rowh.at[pl.ds(0, CHUNK)], rowb[b], si[b]).wait()

        def g_start(b):
            pltpu.async_copy(th.at[colb[b]], gc[b], sg[b])
            pltpu.async_copy(th.at[rowb[b]], gr[b], sg[b])

        def g_wait(b):
            pltpu.make_async_copy(th.at[colb[b]], gc[b], sg[b]).wait()
            pltpu.make_async_copy(th.at[rowb[b]], gr[b], sg[b]).wait()

        def w_start(g, b):
            base = obase0 + g * CHUNK
            for o, oref in zip(outs, ob[b]):
                pltpu.async_copy(oref, o.at[pl.ds(base, CHUNK)], sw[b])

        def w_wait(b):
            for o, oref in zip(outs, ob[b]):
                pltpu.make_async_copy(oref, o.at[pl.ds(0, CHUNK)], sw[b]).wait()

        def combine(b):
            gcb, grb = gc[b], gr[b]
            o1b, o2b = ob[b]

            @plsc.parallel_loop(0, CHUNK, 1, unroll=4)
            def _(i):
                for j in range(H // 16):
                    sl = pl.ds(j * 16, 16)
                    sh = pl.ds(H + j * 16, 16)
                    o1b[i, sl] = gcb[i, sl] + grb[i, sh]
                    o2b[i, sl] = grb[i, sl] + gcb[i, sh]

        idx_start(0, 0)
        idx_wait(0)
        g_start(0)
        idx_start(1, 1)

        def pair(t, _):
            for b in range(2):
                g = 2 * t + b
                nb_ = 1 - b

                @pl.when(g < NCHUNK)
                def _():
                    g_wait(b)

                    @pl.when(g + 1 < NCHUNK)
                    def _():
                        idx_wait(nb_)
                        g_start(nb_)

                    @pl.when(g + 2 < NCHUNK)
                    def _():
                        idx_start(g + 2, b)

                    @pl.when(g >= 2)
                    def _():
                        w_wait(b)

                    combine(b)
                    w_start(g, b)
            return 0

        lax.fori_loop(0, NPAIR, pair, 0)
        w_wait(0)
        @pl.when(NCHUNK > 1)
        def _():
            w_wait(1)

    return pl.kernel(body, out_type=out_type, mesh=_MESH, scratch_types=scratch)


def _make_gather_one(seg):
    seg_base = seg * ESEG
    out_type = jax.ShapeDtypeStruct((ESEG, H), _f32)
    scratch = (
        [pltpu.VMEM((CHUNK,), _i32)] * 4 +
        [pltpu.VMEM((CHUNK, H), _f32)] * 4 +
        [pltpu.VMEM((CHUNK, H), _f32)] * 2 +
        [pltpu.SemaphoreType.DMA] * 6
    )

    def body(tph, tqh, colh, rowh, dummy, o1h,
             colb0, colb1, rowb0, rowb1, gp0, gp1, gq0, gq1,
             o10, o11, si0, si1, sg0, sg1, sw0, sw1):
        del dummy
        colb = (colb0, colb1)
        rowb = (rowb0, rowb1)
        gp = (gp0, gp1)
        gq = (gq0, gq1)
        ob = (o10, o11)
        si = (si0, si1)
        sg = (sg0, sg1)
        sw = (sw0, sw1)
        obase0 = _wid() * EPW
        base0 = seg_base + obase0

        def idx_start(g, b):
            base = base0 + g * CHUNK
            pltpu.async_copy(colh.at[pl.ds(base, CHUNK)], colb[b], si[b])
            pltpu.async_copy(rowh.at[pl.ds(base, CHUNK)], rowb[b], si[b])

        def idx_wait(b):
            pltpu.make_async_copy(colh.at[pl.ds(0, CHUNK)], colb[b], si[b]).wait()
            pltpu.make_async_copy(rowh.at[pl.ds(0, CHUNK)], rowb[b], si[b]).wait()

        def g_start(b):
            pltpu.async_copy(tph.at[colb[b]], gp[b], sg[b])
            pltpu.async_copy(tqh.at[rowb[b]], gq[b], sg[b])

        def g_wait(b):
            pltpu.make_async_copy(tph.at[colb[b]], gp[b], sg[b]).wait()
            pltpu.make_async_copy(tqh.at[rowb[b]], gq[b], sg[b]).wait()

        def w_start(g, b):
            base = obase0 + g * CHUNK
            pltpu.async_copy(ob[b], o1h.at[pl.ds(base, CHUNK)], sw[b])

        def w_wait(b):
            pltpu.make_async_copy(ob[b], o1h.at[pl.ds(0, CHUNK)], sw[b]).wait()

        def combine(b):
            gpb, gqb = gp[b], gq[b]
            o1b = ob[b]

            @plsc.parallel_loop(0, CHUNK, 1, unroll=4)
            def _(i):
                for j in range(H // 16):
                    sl = pl.ds(j * 16, 16)
                    o1b[i, sl] = gpb[i, sl] + gqb[i, sl]

        idx_start(0, 0)
        idx_wait(0)
        g_start(0)
        idx_start(1, 1)

        def pair(t, _):
            for b in range(2):
                g = 2 * t + b
                nb_ = 1 - b

                @pl.when(g < NCHUNK)
                def _():
                    g_wait(b)

                    @pl.when(g + 1 < NCHUNK)
                    def _():
                        idx_wait(nb_)
                        g_start(nb_)

                    @pl.when(g + 2 < NCHUNK)
                    def _():
                        idx_start(g + 2, b)

                    @pl.when(g >= 2)
                    def _():
                        w_wait(b)

                    combine(b)
                    w_start(g, b)
            return 0

        lax.fori_loop(0, NPAIR, pair, 0)
        w_wait(0)
        @pl.when(NCHUNK > 1)
        def _():
            w_wait(1)

    return pl.kernel(body, out_type=out_type, mesh=_MESH, scratch_types=scratch)


def _scatter_body(*refs):
    msgs = refs[:SEG]
    (colh, dummy, outh, colb0, colb1, mb0, mb1, zb, shared,
     si0, si1, sc0, sc1) = refs[SEG:]
    del dummy
    cid = lax.axis_index("c")
    sid = lax.axis_index("s")
    wid = sid * NCORE + cid
    colb = (colb0, colb1)
    mb = (mb0, mb1)
    si = (si0, si1)
    sc = (sc0, sc1)
    zero = jnp.zeros((16,), _f32)

    def zrow(i, _):
        for j in range(H // 16):
            zb[i, pl.ds(j * 16, 16)] = zero
        return 0

    lax.fori_loop(0, ZROWS, zrow, 0)
    for t in range(ROWS_PER_SUB // ZROWS):
        pltpu.sync_copy(zb, shared.at[pl.ds(sid * ROWS_PER_SUB + t * ZROWS, ZROWS)])
    plsc.subcore_barrier()

    base0 = wid * EPW
    for s in range(SEG):
        msgh = msgs[s]
        seg_base = s * ESEG

        def in_start(g, b):
            base = base0 + g * CHUNK
            pltpu.async_copy(colh.at[pl.ds(seg_base + base, CHUNK)], colb[b], si[b])
            pltpu.async_copy(msgh.at[pl.ds(base, CHUNK)], mb[b], si[b])

        def in_wait(b):
            pltpu.make_async_copy(colh.at[pl.ds(0, CHUNK)], colb[b], si[b]).wait()
            pltpu.make_async_copy(msgh.at[pl.ds(0, CHUNK)], mb[b], si[b]).wait()

        def sc_start(b):
            pltpu.async_copy(mb[b], shared.at[colb[b]], sc[b], add=True)

        def sc_wait(b):
            pltpu.make_async_copy(mb[b], shared.at[colb[b]], sc[b]).wait()

        in_start(0, 0)

        def pair(t, _):
            for b in range(2):
                g = 2 * t + b
                nb_ = 1 - b

                @pl.when(g < NCHUNK)
                def _():
                    in_wait(b)
                    sc_start(b)

                    @pl.when(g >= 1)
                    def _():
                        sc_wait(nb_)

                    @pl.when(g + 1 < NCHUNK)
                    def _():
                        in_start(g + 1, nb_)
            return 0

        lax.fori_loop(0, NPAIR, pair, 0)
        sc_wait((NCHUNK - 1) % 2)
    plsc.subcore_barrier()
    for t in range(ROWS_PER_SUB // ZROWS):
        r0 = sid * ROWS_PER_SUB + t * ZROWS
        pltpu.sync_copy(shared.at[pl.ds(r0, ZROWS)], zb)
        pltpu.sync_copy(zb, outh.at[cid, pl.ds(r0, ZROWS)])


_scatter_all = pl.kernel(
    _scatter_body,
    out_type=_AG_SHAPE,
    mesh=_MESH,
    scratch_types=[
        pltpu.VMEM((CHUNK,), _i32),
        pltpu.VMEM((CHUNK,), _i32),
        pltpu.VMEM((CHUNK, H), _f32),
        pltpu.VMEM((CHUNK, H), _f32),
        pltpu.VMEM((ZROWS, H), _f32),
        pltpu.VMEM_SHARED((NPAD, H), _f32),
        pltpu.SemaphoreType.DMA,
        pltpu.SemaphoreType.DMA,
        pltpu.SemaphoreType.DMA,
        pltpu.SemaphoreType.DMA,
    ],
)


_gather_both_k = [_make_gather_both(s) for s in range(SEG)]
_gather_one_k = [_make_gather_one(s) for s in range(SEG)]


def _tok(a):
    if a.ndim == 3:
        return a[0, 0, :8]
    return a[0, :8]


def kernel(x, edge_index, edge_attr, params):
    row = edge_index[0].astype(_i32)
    col = edge_index[1].astype(_i32)
    n0 = params['nodeMLP0']
    e0 = params['edgeMLP0']
    ge = params['gnn_edge']
    gn = params['gnn_node']
    fin = params['final']
    W1p = ge['W1'][:H]
    W1q = ge['W1'][H:2 * H]
    W1pq = jnp.concatenate([W1p, W1q], axis=1)
    W1c = ge['W1'][2 * H:]
    Wna = gn['W1'][:H]
    Wnb = gn['W1'][H:]

    x_, tpq = _node0_call(x, n0['W1'], n0['b1'], n0['W2'], n0['b2'],
                          n0['g'], n0['be'], W1pq)
    cs = [_edge0_call(edge_attr, s, e0['W1'], e0['b1'], e0['W2'], e0['b2'],
                      e0['g'], e0['be'], W1c, ge['b1'])
          for s in range(SEG)]

    tp = tq = None
    out = None
    tok = jnp.zeros((8,), _f32)
    for r in range(3):
        pres = [None] * SEG
        ncs = [None] * SEG

        def do_gather(s, tk):
            if r < 2:
                p = _gather_both_k[s](tpq, col, row, tk)
                return p, _tok(p[0])
            p = _gather_one_k[s](tp, tq, col, row, tk)
            return p, _tok(p)

        def do_edge(s):
            if r < 2:
                msg, cn = _edge_main_call(pres[s][0], pres[s][1], cs[s],
                                          ge['W2'], ge['b2'], ge['g'], ge['be'],
                                          W1c, ge['b1'])
                ncs[s] = cn
                return msg
            return _edge_last_call(pres[s], cs[s], ge['W2'], ge['b2'],
                                   ge['g'], ge['be'])

        msgs = [None] * SEG
        for s in range(SEG):
            pres[s], tok = do_gather(s, tok)
            if s >= 1:
                msgs[s - 1] = do_edge(s - 1)
        msgs[SEG - 1] = do_edge(SEG - 1)
        ag = _scatter_all(*msgs, col, tok)
        tok = _tok(ag)
        if r < 2:
            cs = ncs
        if r == 0:
            x_, tpq = _node_upd_call(ag, x_, Wna, Wnb, gn['b1'],
                                     gn['W2'], gn['b2'], gn['g'], gn['be'], W1pq)
        elif r == 1:
            x_, tp, tq = _node_upd_split_call(ag, x_, Wna, Wnb, gn['b1'],
                                              gn['W2'], gn['b2'], gn['g'],
                                              gn['be'], W1p, W1q)
        else:
            out = _node_fin_call(ag, x_, Wna, Wnb, gn['b1'],
                                 gn['W2'], gn['b2'], gn['g'], gn['be'],
                                 fin['W1'], fin['b1'], fin['W2'], fin['b2'])
    return out


# speedup vs baseline: 1.0446x; 1.0446x over previous
import jax
import jax.numpy as jnp
from jax import lax
from jax.experimental import pallas as pl
from jax.experimental.pallas import tpu as pltpu
from jax.experimental.pallas import tpu_sc as plsc

N = 10000
E = 320000
H = 128
H2 = 2 * H
NCORE = 2
NSUB = 16
NW = NCORE * NSUB
SEG = 5
ESEG = E // SEG
EPW = ESEG // NW
CHUNK = 80
NCHUNK = EPW // CHUNK
NPAIR = (NCHUNK + 1) // 2
NPAD = 10240
ROWS_PER_SUB = NPAD // NSUB
ZROWS = 128

_f32 = jnp.float32
_i32 = jnp.int32


def _mm(a, b):
    return jnp.dot(a, b, preferred_element_type=_f32)


def _ln(h, g, be):
    m = jnp.mean(h, axis=-1, keepdims=True)
    d = h - m
    v = jnp.mean(d * d, axis=-1, keepdims=True)
    return d * lax.rsqrt(v + 1e-5) * g + be


def _full(shape):
    nd = len(shape)
    return pl.BlockSpec(shape, lambda i, _nd=nd: (0,) * _nd)


NBLK = 2000
EBLK = 2000
SBLKS = ESEG // EBLK


def _node0_body(x, W1, b1, W2, b2, g, be, W1pq, xo, to):
    h = jnp.maximum(_mm(x[...], W1[...]) + b1[...][None, :], 0.0)
    h = jnp.maximum(_mm(h, W2[...]) + b2[...][None, :], 0.0)
    xn = _ln(h, g[...][None, :], be[...][None, :])
    xo[...] = xn
    to[...] = _mm(xn, W1pq[...])


def _edge0_body(ea, W1, b1, W2, b2, g, be, W1c, b1e, co):
    h = jnp.maximum(_mm(ea[...], W1[...]) + b1[...][None, :], 0.0)
    h = jnp.maximum(_mm(h, W2[...]) + b2[...][None, :], 0.0)
    e0 = _ln(h, g[...][None, :], be[...][None, :])
    co[...] = _mm(e0, W1c[...]) + b1e[...][None, :]


def _edge_main_body(p1, p2, c, W2, b2, g, be, W1c, b1e, mo, co):
    cc = c[...]
    h = jnp.maximum(_mm(jnp.maximum(p1[...] + cc, 0.0), W2[...]) + b2[...][None, :], 0.0)
    mo[...] = _ln(h, g[...][None, :], be[...][None, :])
    h2 = jnp.maximum(_mm(jnp.maximum(p2[...] + cc, 0.0), W2[...]) + b2[...][None, :], 0.0)
    t = _ln(h2, g[...][None, :], be[...][None, :])
    co[...] = _mm(t, W1c[...]) + b1e[...][None, :]


def _edge_last_body(p1, c, W2, b2, g, be, mo):
    h = jnp.maximum(_mm(jnp.maximum(p1[...] + c[...], 0.0), W2[...]) + b2[...][None, :], 0.0)
    mo[...] = _ln(h, g[...][None, :], be[...][None, :])


def _node_upd_body(agA, agB, x, Wna, Wnb, b1n, W2n, b2n, gn, ben, W1pq, xo, to):
    a = (agA[...][0] + agA[...][1]) + (agB[...][0] + agB[...][1])
    pre = _mm(a, Wna[...]) + _mm(x[...], Wnb[...]) + b1n[...][None, :]
    h = jnp.maximum(_mm(jnp.maximum(pre, 0.0), W2n[...]) + b2n[...][None, :], 0.0)
    xn = _ln(h, gn[...][None, :], ben[...][None, :])
    xo[...] = xn
    to[...] = _mm(xn, W1pq[...])


def _node_upd_split_body(agA, agB, x, Wna, Wnb, b1n, W2n, b2n, gn, ben,
                         W1p, W1q, xo, tpo, tqo):
    a = (agA[...][0] + agA[...][1]) + (agB[...][0] + agB[...][1])
    pre = _mm(a, Wna[...]) + _mm(x[...], Wnb[...]) + b1n[...][None, :]
    h = jnp.maximum(_mm(jnp.maximum(pre, 0.0), W2n[...]) + b2n[...][None, :], 0.0)
    xn = _ln(h, gn[...][None, :], ben[...][None, :])
    xo[...] = xn
    tpo[...] = _mm(xn, W1p[...])
    tqo[...] = _mm(xn, W1q[...])


def _node_fin_body(agA, agB, x, Wna, Wnb, b1n, W2n, b2n, gn, ben, Wf1, bf1,
                   Wf2, bf2, oo):
    a = (agA[...][0] + agA[...][1]) + (agB[...][0] + agB[...][1])
    pre = _mm(a, Wna[...]) + _mm(x[...], Wnb[...]) + b1n[...][None, :]
    h = jnp.maximum(_mm(jnp.maximum(pre, 0.0), W2n[...]) + b2n[...][None, :], 0.0)
    xn = _ln(h, gn[...][None, :], ben[...][None, :])
    h2 = jnp.maximum(_mm(xn, Wf1[...]) + bf1[...][None, :], 0.0)
    oo[...] = _mm(h2, Wf2[...]) + bf2[...][None, :]


def _row_spec(blk, width):
    return pl.BlockSpec((blk, width), lambda i: (i, 0))


def _seg_spec(blk, width, seg):
    off = seg * SBLKS
    return pl.BlockSpec((blk, width), lambda i, _o=off: (_o + i, 0))


_AG_SPEC = pl.BlockSpec((2, NBLK, H), lambda i: (0, i, 0))
_AG_SHAPE = jax.ShapeDtypeStruct((NCORE, NPAD, H), _f32)


def _node0_call(x, W1, b1, W2, b2, g, be, W1pq):
    nb = N // NBLK
    wspecs = [_full(w.shape) for w in (W1, b1, W2, b2, g, be, W1pq)]
    return pl.pallas_call(
        _node0_body,
        grid=(nb,),
        in_specs=[_row_spec(NBLK, H)] + wspecs,
        out_specs=[_row_spec(NBLK, H), _row_spec(NBLK, H2)],
        out_shape=[jax.ShapeDtypeStruct((N, H), _f32),
                   jax.ShapeDtypeStruct((N, H2), _f32)],
    )(x, W1, b1, W2, b2, g, be, W1pq)


def _edge0_call(ea, seg, W1, b1, W2, b2, g, be, W1c, b1e):
    ws = (W1, b1, W2, b2, g, be, W1c, b1e)
    wspecs = [_full(w.shape) for w in ws]
    return pl.pallas_call(
        _edge0_body,
        grid=(SBLKS,),
        in_specs=[_seg_spec(EBLK, ea.shape[1], seg)] + wspecs,
        out_specs=_row_spec(EBLK, H),
        out_shape=jax.ShapeDtypeStruct((ESEG, H), _f32),
    )(ea, *ws)


def _edge_main_call(p1, p2, c, W2, b2, g, be, W1c, b1e):
    ws = (W2, b2, g, be, W1c, b1e)
    wspecs = [_full(w.shape) for w in ws]
    return pl.pallas_call(
        _edge_main_body,
        grid=(SBLKS,),
        in_specs=[_row_spec(EBLK, H)] * 3 + wspecs,
        out_specs=[_row_spec(EBLK, H)] * 2,
        out_shape=[jax.ShapeDtypeStruct((ESEG, H), _f32)] * 2,
    )(p1, p2, c, *ws)


def _edge_last_call(p1, c, W2, b2, g, be):
    ws = (W2, b2, g, be)
    wspecs = [_full(w.shape) for w in ws]
    return pl.pallas_call(
        _edge_last_body,
        grid=(SBLKS,),
        in_specs=[_row_spec(EBLK, H)] * 2 + wspecs,
        out_specs=_row_spec(EBLK, H),
        out_shape=jax.ShapeDtypeStruct((ESEG, H), _f32),
    )(p1, c, *ws)


def _node_upd_call(agA, agB, x, Wna, Wnb, b1n, W2n, b2n, gn, ben, W1pq):
    nb = N // NBLK
    ws = (Wna, Wnb, b1n, W2n, b2n, gn, ben, W1pq)
    wspecs = [_full(w.shape) for w in ws]
    return pl.pallas_call(
        _node_upd_body,
        grid=(nb,),
        in_specs=[_AG_SPEC, _AG_SPEC, _row_spec(NBLK, H)] + wspecs,
        out_specs=[_row_spec(NBLK, H), _row_spec(NBLK, H2)],
        out_shape=[jax.ShapeDtypeStruct((N, H), _f32),
                   jax.ShapeDtypeStruct((N, H2), _f32)],
    )(agA, agB, x, *ws)


def _node_upd_split_call(agA, agB, x, Wna, Wnb, b1n, W2n, b2n, gn, ben, W1p, W1q):
    nb = N // NBLK
    ws = (Wna, Wnb, b1n, W2n, b2n, gn, ben, W1p, W1q)
    wspecs = [_full(w.shape) for w in ws]
    return pl.pallas_call(
        _node_upd_split_body,
        grid=(nb,),
        in_specs=[_AG_SPEC, _AG_SPEC, _row_spec(NBLK, H)] + wspecs,
        out_specs=[_row_spec(NBLK, H)] * 3,
        out_shape=[jax.ShapeDtypeStruct((N, H), _f32)] * 3,
    )(agA, agB, x, *ws)


def _node_fin_call(agA, agB, x, Wna, Wnb, b1n, W2n, b2n, gn, ben, Wf1, bf1, Wf2, bf2):
    nb = N // NBLK
    ws = (Wna, Wnb, b1n, W2n, b2n, gn, ben, Wf1, bf1, Wf2, bf2)
    wspecs = [_full(w.shape) for w in ws]
    dout = Wf2.shape[1]
    return pl.pallas_call(
        _node_fin_body,
        grid=(nb,),
        in_specs=[_AG_SPEC, _AG_SPEC, _row_spec(NBLK, H)] + wspecs,
        out_specs=_row_spec(NBLK, dout),
        out_shape=jax.ShapeDtypeStruct((N, dout), _f32),
    )(agA, agB, x, *ws)


_MESH = plsc.VectorSubcoreMesh(core_axis_name="c", subcore_axis_name="s")


def _wid():
    return lax.axis_index("s") * NCORE + lax.axis_index("c")


def _make_gather_both(seg):
    seg_base = seg * ESEG
    out_type = (jax.ShapeDtypeStruct((ESEG, H), _f32),
                jax.ShapeDtypeStruct((ESEG, H), _f32))
    scratch = (
        [pltpu.VMEM((CHUNK,), _i32)] * 4 +
        [pltpu.VMEM((CHUNK, H2), _f32)] * 4 +
        [pltpu.VMEM((CHUNK, H), _f32)] * 4 +
        [pltpu.SemaphoreType.DMA] * 6
    )

    def body(th, colh, rowh, dummy, o1h, o2h,
             colb0, colb1, rowb0, rowb1, gc0, gc1, gr0, gr1,
             o10, o20, o11, o21, si0, si1, sg0, sg1, sw0, sw1):
        del dummy
        colb = (colb0, colb1)
        rowb = (rowb0, rowb1)
        gc = (gc0, gc1)
        gr = (gr0, gr1)
        ob = ((o10, o20), (o11, o21))
        si = (si0, si1)
        sg = (sg0, sg1)
        sw = (sw0, sw1)
        outs = (o1h, o2h)
        obase0 = _wid() * EPW
        base0 = seg_base + obase0

        def idx_start(g, b):
            base = base0 + g * CHUNK
            pltpu.async_copy(colh.at[pl.ds(base, CHUNK)], colb[b], si[b])
            pltpu.async_copy(rowh.at[pl.ds(base, CHUNK)], rowb[b], si[b])

        def idx_wait(b):
            pltpu.make_async_copy(colh.at[pl.ds(0, CHUNK)], colb[b], si[b]).wait()
            pltpu.make_async_copy(rowh.at[pl.ds(0, CHUNK)], rowb[b], si[b]).wait()

        def g_start(b):
            pltpu.async_copy(th.at[colb[b]], gc[b], sg[b])
            pltpu.async_copy(th.at[rowb[b]], gr[b], sg[b])

        def g_wait(b):
            pltpu.make_async_copy(th.at[colb[b]], gc[b], sg[b]).wait()
            pltpu.make_async_copy(th.at[rowb[b]], gr[b], sg[b]).wait()

        def w_start(g, b):
            base = obase0 + g * CHUNK
            for o, oref in zip(outs, ob[b]):
                pltpu.async_copy(oref, o.at[pl.ds(base, CHUNK)], sw[b])

        def w_wait(b):
            for o, oref in zip(outs, ob[b]):
                pltpu.make_async_copy(oref, o.at[pl.ds(0, CHUNK)], sw[b]).wait()

        def combine(b):
            gcb, grb = gc[b], gr[b]
            o1b, o2b = ob[b]

            @plsc.parallel_loop(0, CHUNK, 1, unroll=4)
            def _(i):
                for j in range(H // 16):
                    sl = pl.ds(j * 16, 16)
                    sh = pl.ds(H + j * 16, 16)
                    o1b[i, sl] = gcb[i, sl] + grb[i, sh]
                    o2b[i, sl] = grb[i, sl] + gcb[i, sh]

        idx_start(0, 0)
        idx_wait(0)
        g_start(0)
        idx_start(1, 1)

        def pair(t, _):
            for b in range(2):
                g = 2 * t + b
                nb_ = 1 - b

                @pl.when(g < NCHUNK)
                def _():
                    g_wait(b)

                    @pl.when(g + 1 < NCHUNK)
                    def _():
                        idx_wait(nb_)
                        g_start(nb_)

                    @pl.when(g + 2 < NCHUNK)
                    def _():
                        idx_start(g + 2, b)

                    @pl.when(g >= 2)
                    def _():
                        w_wait(b)

                    combine(b)
                    w_start(g, b)
            return 0

        lax.fori_loop(0, NPAIR, pair, 0)
        w_wait(0)
        @pl.when(NCHUNK > 1)
        def _():
            w_wait(1)

    return pl.kernel(body, out_type=out_type, mesh=_MESH, scratch_types=scratch)


def _make_gather_one(seg):
    seg_base = seg * ESEG
    out_type = jax.ShapeDtypeStruct((ESEG, H), _f32)
    scratch = (
        [pltpu.VMEM((CHUNK,), _i32)] * 4 +
        [pltpu.VMEM((CHUNK, H), _f32)] * 4 +
        [pltpu.VMEM((CHUNK, H), _f32)] * 2 +
        [pltpu.SemaphoreType.DMA] * 6
    )

    def body(tph, tqh, colh, rowh, dummy, o1h,
             colb0, colb1, rowb0, rowb1, gp0, gp1, gq0, gq1,
             o10, o11, si0, si1, sg0, sg1, sw0, sw1):
        del dummy
        colb = (colb0, colb1)
        rowb = (rowb0, rowb1)
        gp = (gp0, gp1)
        gq = (gq0, gq1)
        ob = (o10, o11)
        si = (si0, si1)
        sg = (sg0, sg1)
        sw = (sw0, sw1)
        obase0 = _wid() * EPW
        base0 = seg_base + obase0

        def idx_start(g, b):
            base = base0 + g * CHUNK
            pltpu.async_copy(colh.at[pl.ds(base, CHUNK)], colb[b], si[b])
            pltpu.async_copy(rowh.at[pl.ds(base, CHUNK)], rowb[b], si[b])

        def idx_wait(b):
            pltpu.make_async_copy(colh.at[pl.ds(0, CHUNK)], colb[b], si[b]).wait()
            pltpu.make_async_copy(rowh.at[pl.ds(0, CHUNK)], rowb[b], si[b]).wait()

        def g_start(b):
            pltpu.async_copy(tph.at[colb[b]], gp[b], sg[b])
            pltpu.async_copy(tqh.at[rowb[b]], gq[b], sg[b])

        def g_wait(b):
            pltpu.make_async_copy(tph.at[colb[b]], gp[b], sg[b]).wait()
            pltpu.make_async_copy(tqh.at[rowb[b]], gq[b], sg[b]).wait()

        def w_start(g, b):
            base = obase0 + g * CHUNK
            pltpu.async_copy(ob[b], o1h.at[pl.ds(base, CHUNK)], sw[b])

        def w_wait(b):
            pltpu.make_async_copy(ob[b], o1h.at[pl.ds(0, CHUNK)], sw[b]).wait()

        def combine(b):
            gpb, gqb = gp[b], gq[b]
            o1b = ob[b]

            @plsc.parallel_loop(0, CHUNK, 1, unroll=4)
            def _(i):
                for j in range(H // 16):
                    sl = pl.ds(j * 16, 16)
                    o1b[i, sl] = gpb[i, sl] + gqb[i, sl]

        idx_start(0, 0)
        idx_wait(0)
        g_start(0)
        idx_start(1, 1)

        def pair(t, _):
            for b in range(2):
                g = 2 * t + b
                nb_ = 1 - b

                @pl.when(g < NCHUNK)
                def _():
                    g_wait(b)

                    @pl.when(g + 1 < NCHUNK)
                    def _():
                        idx_wait(nb_)
                        g_start(nb_)

                    @pl.when(g + 2 < NCHUNK)
                    def _():
                        idx_start(g + 2, b)

                    @pl.when(g >= 2)
                    def _():
                        w_wait(b)

                    combine(b)
                    w_start(g, b)
            return 0

        lax.fori_loop(0, NPAIR, pair, 0)
        w_wait(0)
        @pl.when(NCHUNK > 1)
        def _():
            w_wait(1)

    return pl.kernel(body, out_type=out_type, mesh=_MESH, scratch_types=scratch)


def _make_scatter_multi(seg_list):
  nseg = len(seg_list)

  def _scatter_body(*refs):
    msgs = refs[:nseg]
    (colh, dummy, outh, colb0, colb1, mb0, mb1, zb, shared,
     si0, si1, sc0, sc1) = refs[nseg:]
    del dummy
    cid = lax.axis_index("c")
    sid = lax.axis_index("s")
    wid = sid * NCORE + cid
    colb = (colb0, colb1)
    mb = (mb0, mb1)
    si = (si0, si1)
    sc = (sc0, sc1)
    zero = jnp.zeros((16,), _f32)

    def zrow(i, _):
        for j in range(H // 16):
            zb[i, pl.ds(j * 16, 16)] = zero
        return 0

    lax.fori_loop(0, ZROWS, zrow, 0)
    for t in range(ROWS_PER_SUB // ZROWS):
        pltpu.sync_copy(zb, shared.at[pl.ds(sid * ROWS_PER_SUB + t * ZROWS, ZROWS)])
    plsc.subcore_barrier()

    base0 = wid * EPW
    for si_, s in enumerate(seg_list):
        msgh = msgs[si_]
        seg_base = s * ESEG

        def in_start(g, b):
            base = base0 + g * CHUNK
            pltpu.async_copy(colh.at[pl.ds(seg_base + base, CHUNK)], colb[b], si[b])
            pltpu.async_copy(msgh.at[pl.ds(base, CHUNK)], mb[b], si[b])

        def in_wait(b):
            pltpu.make_async_copy(colh.at[pl.ds(0, CHUNK)], colb[b], si[b]).wait()
            pltpu.make_async_copy(msgh.at[pl.ds(0, CHUNK)], mb[b], si[b]).wait()

        def sc_start(b):
            pltpu.async_copy(mb[b], shared.at[colb[b]], sc[b], add=True)

        def sc_wait(b):
            pltpu.make_async_copy(mb[b], shared.at[colb[b]], sc[b]).wait()

        in_start(0, 0)

        def pair(t, _):
            for b in range(2):
                g = 2 * t + b
                nb_ = 1 - b

                @pl.when(g < NCHUNK)
                def _():
                    in_wait(b)
                    sc_start(b)

                    @pl.when(g >= 1)
                    def _():
                        sc_wait(nb_)

                    @pl.when(g + 1 < NCHUNK)
                    def _():
                        in_start(g + 1, nb_)
            return 0

        lax.fori_loop(0, NPAIR, pair, 0)
        sc_wait((NCHUNK - 1) % 2)
    plsc.subcore_barrier()
    for t in range(ROWS_PER_SUB // ZROWS):
        r0 = sid * ROWS_PER_SUB + t * ZROWS
        pltpu.sync_copy(shared.at[pl.ds(r0, ZROWS)], zb)
        pltpu.sync_copy(zb, outh.at[cid, pl.ds(r0, ZROWS)])


  return pl.kernel(
      _scatter_body,
      out_type=_AG_SHAPE,
      mesh=_MESH,
      scratch_types=[
          pltpu.VMEM((CHUNK,), _i32),
          pltpu.VMEM((CHUNK,), _i32),
          pltpu.VMEM((CHUNK, H), _f32),
          pltpu.VMEM((CHUNK, H), _f32),
          pltpu.VMEM((ZROWS, H), _f32),
          pltpu.VMEM_SHARED((NPAD, H), _f32),
          pltpu.SemaphoreType.DMA,
          pltpu.SemaphoreType.DMA,
          pltpu.SemaphoreType.DMA,
          pltpu.SemaphoreType.DMA,
      ],
  )


_scatter_head = _make_scatter_multi(list(range(SEG - 1)))
_scatter_tail = _make_scatter_multi([SEG - 1])


_gather_both_k = [_make_gather_both(s) for s in range(SEG)]
_gather_one_k = [_make_gather_one(s) for s in range(SEG)]


def _tok(a):
    if a.ndim == 3:
        return a[0, 0, :8]
    return a[0, :8]


def kernel(x, edge_index, edge_attr, params):
    row = edge_index[0].astype(_i32)
    col = edge_index[1].astype(_i32)
    n0 = params['nodeMLP0']
    e0 = params['edgeMLP0']
    ge = params['gnn_edge']
    gn = params['gnn_node']
    fin = params['final']
    W1p = ge['W1'][:H]
    W1q = ge['W1'][H:2 * H]
    W1pq = jnp.concatenate([W1p, W1q], axis=1)
    W1c = ge['W1'][2 * H:]
    Wna = gn['W1'][:H]
    Wnb = gn['W1'][H:]

    x_, tpq = _node0_call(x, n0['W1'], n0['b1'], n0['W2'], n0['b2'],
                          n0['g'], n0['be'], W1pq)
    cs = [_edge0_call(edge_attr, s, e0['W1'], e0['b1'], e0['W2'], e0['b2'],
                      e0['g'], e0['be'], W1c, ge['b1'])
          for s in range(SEG)]

    tp = tq = None
    out = None
    tok = jnp.zeros((8,), _f32)
    for r in range(3):
        pres = [None] * SEG
        ncs = [None] * SEG

        def do_gather(s, tk):
            if r < 2:
                p = _gather_both_k[s](tpq, col, row, tk)
                return p, _tok(p[0])
            p = _gather_one_k[s](tp, tq, col, row, tk)
            return p, _tok(p)

        def do_edge(s):
            if r < 2:
                msg, cn = _edge_main_call(pres[s][0], pres[s][1], cs[s],
                                          ge['W2'], ge['b2'], ge['g'], ge['be'],
                                          W1c, ge['b1'])
                ncs[s] = cn
                return msg
            return _edge_last_call(pres[s], cs[s], ge['W2'], ge['b2'],
                                   ge['g'], ge['be'])

        msgs = [None] * SEG
        for s in range(SEG):
            pres[s], tok = do_gather(s, tok)
            if s >= 1:
                msgs[s - 1] = do_edge(s - 1)
        msgs[SEG - 1] = do_edge(SEG - 1)
        agA = _scatter_head(*msgs[:SEG - 1], col, tok)
        tok = _tok(agA)
        agB = _scatter_tail(msgs[SEG - 1], col, tok)
        tok = _tok(agB)
        if r < 2:
            cs = ncs
        if r == 0:
            x_, tpq = _node_upd_call(agA, agB, x_, Wna, Wnb, gn['b1'],
                                     gn['W2'], gn['b2'], gn['g'], gn['be'], W1pq)
        elif r == 1:
            x_, tp, tq = _node_upd_split_call(agA, agB, x_, Wna, Wnb, gn['b1'],
                                              gn['W2'], gn['b2'], gn['g'],
                                              gn['be'], W1p, W1q)
        else:
            out = _node_fin_call(agA, agB, x_, Wna, Wnb, gn['b1'],
                                 gn['W2'], gn['b2'], gn['g'], gn['be'],
                                 fin['W1'], fin['b1'], fin['W2'], fin['b2'])
    return out


# speedup vs baseline: 1.0742x; 1.0283x over previous
import jax
import jax.numpy as jnp
from jax import lax
from jax.experimental import pallas as pl
from jax.experimental.pallas import tpu as pltpu
from jax.experimental.pallas import tpu_sc as plsc

N = 10000
E = 320000
H = 128
H2 = 2 * H
NCORE = 2
NSUB = 16
NW = NCORE * NSUB
SEG = 5
ESEG = E // SEG
EPW = ESEG // NW
CHUNK = 80
NCHUNK = EPW // CHUNK
NPAIR = (NCHUNK + 1) // 2
NPAD = 10240
ROWS_PER_SUB = NPAD // NSUB
ZROWS = 128

_f32 = jnp.float32
_i32 = jnp.int32


def _mm(a, b):
    return jnp.dot(a, b, preferred_element_type=_f32)


def _ln(h, g, be):
    m = jnp.mean(h, axis=-1, keepdims=True)
    d = h - m
    v = jnp.mean(d * d, axis=-1, keepdims=True)
    return d * lax.rsqrt(v + 1e-5) * g + be


def _full(shape):
    nd = len(shape)
    return pl.BlockSpec(shape, lambda i, _nd=nd: (0,) * _nd)


NBLK = 2000
EBLK = 2000
SBLKS = ESEG // EBLK


def _node0_body(x, W1, b1, W2, b2, g, be, W1pq, xo, to):
    h = jnp.maximum(_mm(x[...], W1[...]) + b1[...][None, :], 0.0)
    h = jnp.maximum(_mm(h, W2[...]) + b2[...][None, :], 0.0)
    xn = _ln(h, g[...][None, :], be[...][None, :])
    xo[...] = xn
    to[...] = _mm(xn, W1pq[...])


def _edge0_body(ea, W1, b1, W2, b2, g, be, W1c, b1e, co):
    h = jnp.maximum(_mm(ea[...], W1[...]) + b1[...][None, :], 0.0)
    h = jnp.maximum(_mm(h, W2[...]) + b2[...][None, :], 0.0)
    e0 = _ln(h, g[...][None, :], be[...][None, :])
    co[...] = _mm(e0, W1c[...]) + b1e[...][None, :]


def _edge_main_body(p1, p2, c, W2, b2, g, be, W1c, b1e, mo, co):
    cc = c[...]
    h = jnp.maximum(_mm(jnp.maximum(p1[...] + cc, 0.0), W2[...]) + b2[...][None, :], 0.0)
    mo[...] = _ln(h, g[...][None, :], be[...][None, :])
    h2 = jnp.maximum(_mm(jnp.maximum(p2[...] + cc, 0.0), W2[...]) + b2[...][None, :], 0.0)
    t = _ln(h2, g[...][None, :], be[...][None, :])
    co[...] = _mm(t, W1c[...]) + b1e[...][None, :]


def _edge_last_body(p1, c, W2, b2, g, be, mo):
    h = jnp.maximum(_mm(jnp.maximum(p1[...] + c[...], 0.0), W2[...]) + b2[...][None, :], 0.0)
    mo[...] = _ln(h, g[...][None, :], be[...][None, :])


def _node_upd_body(*refs):
    ags = refs[:SEG]
    (x, Wna, Wnb, b1n, W2n, b2n, gn, ben, W1pq, xo, to) = refs[SEG:]
    a = sum(ag[...][0] + ag[...][1] for ag in ags)
    pre = _mm(a, Wna[...]) + _mm(x[...], Wnb[...]) + b1n[...][None, :]
    h = jnp.maximum(_mm(jnp.maximum(pre, 0.0), W2n[...]) + b2n[...][None, :], 0.0)
    xn = _ln(h, gn[...][None, :], ben[...][None, :])
    xo[...] = xn
    to[...] = _mm(xn, W1pq[...])


def _node_upd_split_body(*refs):
    ags = refs[:SEG]
    (x, Wna, Wnb, b1n, W2n, b2n, gn, ben, W1p, W1q, xo, tpo, tqo) = refs[SEG:]
    a = sum(ag[...][0] + ag[...][1] for ag in ags)
    pre = _mm(a, Wna[...]) + _mm(x[...], Wnb[...]) + b1n[...][None, :]
    h = jnp.maximum(_mm(jnp.maximum(pre, 0.0), W2n[...]) + b2n[...][None, :], 0.0)
    xn = _ln(h, gn[...][None, :], ben[...][None, :])
    xo[...] = xn
    tpo[...] = _mm(xn, W1p[...])
    tqo[...] = _mm(xn, W1q[...])


def _node_fin_body(*refs):
    ags = refs[:SEG]
    (x, Wna, Wnb, b1n, W2n, b2n, gn, ben, Wf1, bf1, Wf2, bf2, oo) = refs[SEG:]
    a = sum(ag[...][0] + ag[...][1] for ag in ags)
    pre = _mm(a, Wna[...]) + _mm(x[...], Wnb[...]) + b1n[...][None, :]
    h = jnp.maximum(_mm(jnp.maximum(pre, 0.0), W2n[...]) + b2n[...][None, :], 0.0)
    xn = _ln(h, gn[...][None, :], ben[...][None, :])
    h2 = jnp.maximum(_mm(xn, Wf1[...]) + bf1[...][None, :], 0.0)
    oo[...] = _mm(h2, Wf2[...]) + bf2[...][None, :]


def _row_spec(blk, width):
    return pl.BlockSpec((blk, width), lambda i: (i, 0))


def _seg_spec(blk, width, seg):
    off = seg * SBLKS
    return pl.BlockSpec((blk, width), lambda i, _o=off: (_o + i, 0))


_AG_SPEC = pl.BlockSpec((2, NBLK, H), lambda i: (0, i, 0))
_AG_SHAPE = jax.ShapeDtypeStruct((NCORE, NPAD, H), _f32)


def _node0_call(x, W1, b1, W2, b2, g, be, W1pq):
    nb = N // NBLK
    wspecs = [_full(w.shape) for w in (W1, b1, W2, b2, g, be, W1pq)]
    return pl.pallas_call(
        _node0_body,
        grid=(nb,),
        in_specs=[_row_spec(NBLK, H)] + wspecs,
        out_specs=[_row_spec(NBLK, H), _row_spec(NBLK, H2)],
        out_shape=[jax.ShapeDtypeStruct((N, H), _f32),
                   jax.ShapeDtypeStruct((N, H2), _f32)],
    )(x, W1, b1, W2, b2, g, be, W1pq)


def _edge0_call(ea, seg, W1, b1, W2, b2, g, be, W1c, b1e):
    ws = (W1, b1, W2, b2, g, be, W1c, b1e)
    wspecs = [_full(w.shape) for w in ws]
    return pl.pallas_call(
        _edge0_body,
        grid=(SBLKS,),
        in_specs=[_seg_spec(EBLK, ea.shape[1], seg)] + wspecs,
        out_specs=_row_spec(EBLK, H),
        out_shape=jax.ShapeDtypeStruct((ESEG, H), _f32),
    )(ea, *ws)


def _edge_main_call(p1, p2, c, W2, b2, g, be, W1c, b1e):
    ws = (W2, b2, g, be, W1c, b1e)
    wspecs = [_full(w.shape) for w in ws]
    return pl.pallas_call(
        _edge_main_body,
        grid=(SBLKS,),
        in_specs=[_row_spec(EBLK, H)] * 3 + wspecs,
        out_specs=[_row_spec(EBLK, H)] * 2,
        out_shape=[jax.ShapeDtypeStruct((ESEG, H), _f32)] * 2,
    )(p1, p2, c, *ws)


def _edge_last_call(p1, c, W2, b2, g, be):
    ws = (W2, b2, g, be)
    wspecs = [_full(w.shape) for w in ws]
    return pl.pallas_call(
        _edge_last_body,
        grid=(SBLKS,),
        in_specs=[_row_spec(EBLK, H)] * 2 + wspecs,
        out_specs=_row_spec(EBLK, H),
        out_shape=jax.ShapeDtypeStruct((ESEG, H), _f32),
    )(p1, c, *ws)


def _node_upd_call(ags, x, Wna, Wnb, b1n, W2n, b2n, gn, ben, W1pq):
    nb = N // NBLK
    ws = (Wna, Wnb, b1n, W2n, b2n, gn, ben, W1pq)
    wspecs = [_full(w.shape) for w in ws]
    return pl.pallas_call(
        _node_upd_body,
        grid=(nb,),
        in_specs=[_AG_SPEC] * SEG + [_row_spec(NBLK, H)] + wspecs,
        out_specs=[_row_spec(NBLK, H), _row_spec(NBLK, H2)],
        out_shape=[jax.ShapeDtypeStruct((N, H), _f32),
                   jax.ShapeDtypeStruct((N, H2), _f32)],
    )(*ags, x, *ws)


def _node_upd_split_call(ags, x, Wna, Wnb, b1n, W2n, b2n, gn, ben, W1p, W1q):
    nb = N // NBLK
    ws = (Wna, Wnb, b1n, W2n, b2n, gn, ben, W1p, W1q)
    wspecs = [_full(w.shape) for w in ws]
    return pl.pallas_call(
        _node_upd_split_body,
        grid=(nb,),
        in_specs=[_AG_SPEC] * SEG + [_row_spec(NBLK, H)] + wspecs,
        out_specs=[_row_spec(NBLK, H)] * 3,
        out_shape=[jax.ShapeDtypeStruct((N, H), _f32)] * 3,
    )(*ags, x, *ws)


def _node_fin_call(ags, x, Wna, Wnb, b1n, W2n, b2n, gn, ben, Wf1, bf1, Wf2, bf2):
    nb = N // NBLK
    ws = (Wna, Wnb, b1n, W2n, b2n, gn, ben, Wf1, bf1, Wf2, bf2)
    wspecs = [_full(w.shape) for w in ws]
    dout = Wf2.shape[1]
    return pl.pallas_call(
        _node_fin_body,
        grid=(nb,),
        in_specs=[_AG_SPEC] * SEG + [_row_spec(NBLK, H)] + wspecs,
        out_specs=_row_spec(NBLK, dout),
        out_shape=jax.ShapeDtypeStruct((N, dout), _f32),
    )(*ags, x, *ws)


_MESH = plsc.VectorSubcoreMesh(core_axis_name="c", subcore_axis_name="s")


def _wid():
    return lax.axis_index("s") * NCORE + lax.axis_index("c")


def _make_gather_both(seg):
    seg_base = seg * ESEG
    out_type = (jax.ShapeDtypeStruct((ESEG, H), _f32),
                jax.ShapeDtypeStruct((ESEG, H), _f32))
    scratch = (
        [pltpu.VMEM((CHUNK,), _i32)] * 4 +
        [pltpu.VMEM((CHUNK, H2), _f32)] * 4 +
        [pltpu.VMEM((CHUNK, H), _f32)] * 4 +
        [pltpu.SemaphoreType.DMA] * 6
    )

    def body(th, colh, rowh, dummy, o1h, o2h,
             colb0, colb1, rowb0, rowb1, gc0, gc1, gr0, gr1,
             o10, o20, o11, o21, si0, si1, sg0, sg1, sw0, sw1):
        del dummy
        colb = (colb0, colb1)
        rowb = (rowb0, rowb1)
        gc = (gc0, gc1)
        gr = (gr0, gr1)
        ob = ((o10, o20), (o11, o21))
        si = (si0, si1)
        sg = (sg0, sg1)
        sw = (sw0, sw1)
        outs = (o1h, o2h)
        obase0 = _wid() * EPW
        base0 = seg_base + obase0

        def idx_start(g, b):
            base = base0 + g * CHUNK
            pltpu.async_copy(colh.at[pl.ds(base, CHUNK)], colb[b], si[b])
            pltpu.async_copy(rowh.at[pl.ds(base, CHUNK)], rowb[b], si[b])

        def idx_wait(b):
            pltpu.make_async_copy(colh.at[pl.ds(0, CHUNK)], colb[b], si[b]).wait()
            pltpu.make_async_copy(rowh.at[pl.ds(0, CHUNK)], rowb[b], si[b]).wait()

        def g_start(b):
            pltpu.async_copy(th.at[colb[b]], gc[b], sg[b])
            pltpu.async_copy(th.at[rowb[b]], gr[b], sg[b])

        def g_wait(b):
            pltpu.make_async_copy(th.at[colb[b]], gc[b], sg[b]).wait()
            pltpu.make_async_copy(th.at[rowb[b]], gr[b], sg[b]).wait()

        def w_start(g, b):
            base = obase0 + g * CHUNK
            for o, oref in zip(outs, ob[b]):
                pltpu.async_copy(oref, o.at[pl.ds(base, CHUNK)], sw[b])

        def w_wait(b):
            for o, oref in zip(outs, ob[b]):
                pltpu.make_async_copy(oref, o.at[pl.ds(0, CHUNK)], sw[b]).wait()

        def combine(b):
            gcb, grb = gc[b], gr[b]
            o1b, o2b = ob[b]

            @plsc.parallel_loop(0, CHUNK, 1, unroll=4)
            def _(i):
                for j in range(H // 16):
                    sl = pl.ds(j * 16, 16)
                    sh = pl.ds(H + j * 16, 16)
                    o1b[i, sl] = gcb[i, sl] + grb[i, sh]
                    o2b[i, sl] = grb[i, sl] + gcb[i, sh]

        idx_start(0, 0)
        idx_wait(0)
        g_start(0)
        idx_start(1, 1)

        def pair(t, _):
            for b in range(2):
                g = 2 * t + b
                nb_ = 1 - b

                @pl.when(g < NCHUNK)
                def _():
                    g_wait(b)

                    @pl.when(g + 1 < NCHUNK)
                    def _():
                        idx_wait(nb_)
                        g_start(nb_)

                    @pl.when(g + 2 < NCHUNK)
                    def _():
                        idx_start(g + 2, b)

                    @pl.when(g >= 2)
                    def _():
                        w_wait(b)

                    combine(b)
                    w_start(g, b)
            return 0

        lax.fori_loop(0, NPAIR, pair, 0)
        w_wait(0)
        @pl.when(NCHUNK > 1)
        def _():
            w_wait(1)

    return pl.kernel(body, out_type=out_type, mesh=_MESH, scratch_types=scratch)


def _make_gather_one(seg):
    seg_base = seg * ESEG
    out_type = jax.ShapeDtypeStruct((ESEG, H), _f32)
    scratch = (
        [pltpu.VMEM((CHUNK,), _i32)] * 4 +
        [pltpu.VMEM((CHUNK, H), _f32)] * 4 +
        [pltpu.VMEM((CHUNK, H), _f32)] * 2 +
        [pltpu.SemaphoreType.DMA] * 6
    )

    def body(tph, tqh, colh, rowh, dummy, o1h,
             colb0, colb1, rowb0, rowb1, gp0, gp1, gq0, gq1,
             o10, o11, si0, si1, sg0, sg1, sw0, sw1):
        del dummy
        colb = (colb0, colb1)
        rowb = (rowb0, rowb1)
        gp = (gp0, gp1)
        gq = (gq0, gq1)
        ob = (o10, o11)
        si = (si0, si1)
        sg = (sg0, sg1)
        sw = (sw0, sw1)
        obase0 = _wid() * EPW
        base0 = seg_base + obase0

        def idx_start(g, b):
            base = base0 + g * CHUNK
            pltpu.async_copy(colh.at[pl.ds(base, CHUNK)], colb[b], si[b])
            pltpu.async_copy(rowh.at[pl.ds(base, CHUNK)], rowb[b], si[b])

        def idx_wait(b):
            pltpu.make_async_copy(colh.at[pl.ds(0, CHUNK)], colb[b], si[b]).wait()
            pltpu.make_async_copy(rowh.at[pl.ds(0, CHUNK)], rowb[b], si[b]).wait()

        def g_start(b):
            pltpu.async_copy(tph.at[colb[b]], gp[b], sg[b])
            pltpu.async_copy(tqh.at[rowb[b]], gq[b], sg[b])

        def g_wait(b):
            pltpu.make_async_copy(tph.at[colb[b]], gp[b], sg[b]).wait()
            pltpu.make_async_copy(tqh.at[rowb[b]], gq[b], sg[b]).wait()

        def w_start(g, b):
            base = obase0 + g * CHUNK
            pltpu.async_copy(ob[b], o1h.at[pl.ds(base, CHUNK)], sw[b])

        def w_wait(b):
            pltpu.make_async_copy(ob[b], o1h.at[pl.ds(0, CHUNK)], sw[b]).wait()

        def combine(b):
            gpb, gqb = gp[b], gq[b]
            o1b = ob[b]

            @plsc.parallel_loop(0, CHUNK, 1, unroll=4)
            def _(i):
                for j in range(H // 16):
                    sl = pl.ds(j * 16, 16)
                    o1b[i, sl] = gpb[i, sl] + gqb[i, sl]

        idx_start(0, 0)
        idx_wait(0)
        g_start(0)
        idx_start(1, 1)

        def pair(t, _):
            for b in range(2):
                g = 2 * t + b
                nb_ = 1 - b

                @pl.when(g < NCHUNK)
                def _():
                    g_wait(b)

                    @pl.when(g + 1 < NCHUNK)
                    def _():
                        idx_wait(nb_)
                        g_start(nb_)

                    @pl.when(g + 2 < NCHUNK)
                    def _():
                        idx_start(g + 2, b)

                    @pl.when(g >= 2)
                    def _():
                        w_wait(b)

                    combine(b)
                    w_start(g, b)
            return 0

        lax.fori_loop(0, NPAIR, pair, 0)
        w_wait(0)
        @pl.when(NCHUNK > 1)
        def _():
            w_wait(1)

    return pl.kernel(body, out_type=out_type, mesh=_MESH, scratch_types=scratch)


def _make_scatter_multi(seg_list):
  nseg = len(seg_list)

  def _scatter_body(*refs):
    msgs = refs[:nseg]
    (colh, dummy, outh, colb0, colb1, mb0, mb1, zb, shared,
     si0, si1, sc0, sc1) = refs[nseg:]
    del dummy
    cid = lax.axis_index("c")
    sid = lax.axis_index("s")
    wid = sid * NCORE + cid
    colb = (colb0, colb1)
    mb = (mb0, mb1)
    si = (si0, si1)
    sc = (sc0, sc1)
    zero = jnp.zeros((16,), _f32)

    def zrow(i, _):
        for j in range(H // 16):
            zb[i, pl.ds(j * 16, 16)] = zero
        return 0

    lax.fori_loop(0, ZROWS, zrow, 0)
    for t in range(ROWS_PER_SUB // ZROWS):
        pltpu.sync_copy(zb, shared.at[pl.ds(sid * ROWS_PER_SUB + t * ZROWS, ZROWS)])
    plsc.subcore_barrier()

    base0 = wid * EPW
    for si_, s in enumerate(seg_list):
        msgh = msgs[si_]
        seg_base = s * ESEG

        def in_start(g, b):
            base = base0 + g * CHUNK
            pltpu.async_copy(colh.at[pl.ds(seg_base + base, CHUNK)], colb[b], si[b])
            pltpu.async_copy(msgh.at[pl.ds(base, CHUNK)], mb[b], si[b])

        def in_wait(b):
            pltpu.make_async_copy(colh.at[pl.ds(0, CHUNK)], colb[b], si[b]).wait()
            pltpu.make_async_copy(msgh.at[pl.ds(0, CHUNK)], mb[b], si[b]).wait()

        def sc_start(b):
            pltpu.async_copy(mb[b], shared.at[colb[b]], sc[b], add=True)

        def sc_wait(b):
            pltpu.make_async_copy(mb[b], shared.at[colb[b]], sc[b]).wait()

        in_start(0, 0)

        def pair(t, _):
            for b in range(2):
                g = 2 * t + b
                nb_ = 1 - b

                @pl.when(g < NCHUNK)
                def _():
                    in_wait(b)
                    sc_start(b)

                    @pl.when(g >= 1)
                    def _():
                        sc_wait(nb_)

                    @pl.when(g + 1 < NCHUNK)
                    def _():
                        in_start(g + 1, nb_)
            return 0

        lax.fori_loop(0, NPAIR, pair, 0)
        sc_wait((NCHUNK - 1) % 2)
    plsc.subcore_barrier()
    for t in range(ROWS_PER_SUB // ZROWS):
        r0 = sid * ROWS_PER_SUB + t * ZROWS
        pltpu.sync_copy(shared.at[pl.ds(r0, ZROWS)], zb)
        pltpu.sync_copy(zb, outh.at[cid, pl.ds(r0, ZROWS)])


  return pl.kernel(
      _scatter_body,
      out_type=_AG_SHAPE,
      mesh=_MESH,
      scratch_types=[
          pltpu.VMEM((CHUNK,), _i32),
          pltpu.VMEM((CHUNK,), _i32),
          pltpu.VMEM((CHUNK, H), _f32),
          pltpu.VMEM((CHUNK, H), _f32),
          pltpu.VMEM((ZROWS, H), _f32),
          pltpu.VMEM_SHARED((NPAD, H), _f32),
          pltpu.SemaphoreType.DMA,
          pltpu.SemaphoreType.DMA,
          pltpu.SemaphoreType.DMA,
          pltpu.SemaphoreType.DMA,
      ],
  )


_scatter_k = [_make_scatter_multi([s]) for s in range(SEG)]


_gather_both_k = [_make_gather_both(s) for s in range(SEG)]
_gather_one_k = [_make_gather_one(s) for s in range(SEG)]


def _tok(a):
    if a.ndim == 3:
        return a[0, 0, :8]
    return a[0, :8]


def kernel(x, edge_index, edge_attr, params):
    row = edge_index[0].astype(_i32)
    col = edge_index[1].astype(_i32)
    n0 = params['nodeMLP0']
    e0 = params['edgeMLP0']
    ge = params['gnn_edge']
    gn = params['gnn_node']
    fin = params['final']
    W1p = ge['W1'][:H]
    W1q = ge['W1'][H:2 * H]
    W1pq = jnp.concatenate([W1p, W1q], axis=1)
    W1c = ge['W1'][2 * H:]
    Wna = gn['W1'][:H]
    Wnb = gn['W1'][H:]

    x_, tpq = _node0_call(x, n0['W1'], n0['b1'], n0['W2'], n0['b2'],
                          n0['g'], n0['be'], W1pq)
    cs = [_edge0_call(edge_attr, s, e0['W1'], e0['b1'], e0['W2'], e0['b2'],
                      e0['g'], e0['be'], W1c, ge['b1'])
          for s in range(SEG)]

    tp = tq = None
    out = None
    tok = jnp.zeros((8,), _f32)
    for r in range(3):
        pres = [None] * SEG
        ncs = [None] * SEG

        def do_gather(s, tk):
            if r < 2:
                p = _gather_both_k[s](tpq, col, row, tk)
                return p, _tok(p[0])
            p = _gather_one_k[s](tp, tq, col, row, tk)
            return p, _tok(p)

        def do_edge(s):
            if r < 2:
                msg, cn = _edge_main_call(pres[s][0], pres[s][1], cs[s],
                                          ge['W2'], ge['b2'], ge['g'], ge['be'],
                                          W1c, ge['b1'])
                ncs[s] = cn
                return msg
            return _edge_last_call(pres[s], cs[s], ge['W2'], ge['b2'],
                                   ge['g'], ge['be'])

        ags = [None] * SEG
        pres[0], tok = do_gather(0, tok)
        for s in range(1, SEG + 1):
            if s < SEG:
                pres[s], tok = do_gather(s, tok)
            msg = do_edge(s - 1)
            ags[s - 1] = _scatter_k[s - 1](msg, col, tok)
            tok = _tok(ags[s - 1])
        if r < 2:
            cs = ncs
        if r == 0:
            x_, tpq = _node_upd_call(ags, x_, Wna, Wnb, gn['b1'],
                                     gn['W2'], gn['b2'], gn['g'], gn['be'], W1pq)
        elif r == 1:
            x_, tp, tq = _node_upd_split_call(ags, x_, Wna, Wnb, gn['b1'],
                                              gn['W2'], gn['b2'], gn['g'],
                                              gn['be'], W1p, W1q)
        else:
            out = _node_fin_call(ags, x_, Wna, Wnb, gn['b1'],
                                 gn['W2'], gn['b2'], gn['g'], gn['be'],
                                 fin['W1'], fin['b1'], fin['W2'], fin['b2'])
    return out


# speedup vs baseline: 1.0756x; 1.0013x over previous
import jax
import jax.numpy as jnp
from jax import lax
from jax.experimental import pallas as pl
from jax.experimental.pallas import tpu as pltpu
from jax.experimental.pallas import tpu_sc as plsc

N = 10000
E = 320000
H = 128
H2 = 2 * H
NCORE = 2
NSUB = 16
NW = NCORE * NSUB
SEG = 5
ESEG = E // SEG
EPW = ESEG // NW
CHUNK = 80
NCHUNK = EPW // CHUNK
NPAIR = (NCHUNK + 1) // 2
NPAD = 10240
ROWS_PER_SUB = NPAD // NSUB
ZROWS = 128

_f32 = jnp.float32
_i32 = jnp.int32


def _mm(a, b):
    return jnp.dot(a, b, preferred_element_type=_f32)


def _ln(h, g, be):
    m = jnp.mean(h, axis=-1, keepdims=True)
    d = h - m
    v = jnp.mean(d * d, axis=-1, keepdims=True)
    return d * lax.rsqrt(v + 1e-5) * g + be


def _full(shape):
    nd = len(shape)
    return pl.BlockSpec(shape, lambda i, _nd=nd: (0,) * _nd)


NBLK = 2000
EBLK = 2000
SBLKS = ESEG // EBLK


def _node0_body(x, W1, b1, W2, b2, g, be, W1pq, xo, to):
    h = jnp.maximum(_mm(x[...], W1[...]) + b1[...][None, :], 0.0)
    h = jnp.maximum(_mm(h, W2[...]) + b2[...][None, :], 0.0)
    xn = _ln(h, g[...][None, :], be[...][None, :])
    xo[...] = xn
    to[...] = _mm(xn, W1pq[...])


def _edge0_body(ea, W1, b1, W2, b2, g, be, W1c, b1e, co):
    h = jnp.maximum(_mm(ea[...], W1[...]) + b1[...][None, :], 0.0)
    h = jnp.maximum(_mm(h, W2[...]) + b2[...][None, :], 0.0)
    e0 = _ln(h, g[...][None, :], be[...][None, :])
    co[...] = _mm(e0, W1c[...]) + b1e[...][None, :]


def _edge_main_body(p1, p2, c, W2, b2, g, be, W1c, b1e, mo, co):
    cc = c[...]
    h = jnp.maximum(_mm(jnp.maximum(p1[...] + cc, 0.0), W2[...]) + b2[...][None, :], 0.0)
    mo[...] = _ln(h, g[...][None, :], be[...][None, :])
    h2 = jnp.maximum(_mm(jnp.maximum(p2[...] + cc, 0.0), W2[...]) + b2[...][None, :], 0.0)
    t = _ln(h2, g[...][None, :], be[...][None, :])
    co[...] = _mm(t, W1c[...]) + b1e[...][None, :]


def _edge_last_body(p1, c, W2, b2, g, be, mo):
    h = jnp.maximum(_mm(jnp.maximum(p1[...] + c[...], 0.0), W2[...]) + b2[...][None, :], 0.0)
    mo[...] = _ln(h, g[...][None, :], be[...][None, :])


def _node_upd_body(*refs):
    ags = refs[:SEG]
    (x, Wna, Wnb, b1n, W2n, b2n, gn, ben, W1pq, xo, to) = refs[SEG:]
    a = sum(ag[...][0] + ag[...][1] for ag in ags)
    pre = _mm(a, Wna[...]) + _mm(x[...], Wnb[...]) + b1n[...][None, :]
    h = jnp.maximum(_mm(jnp.maximum(pre, 0.0), W2n[...]) + b2n[...][None, :], 0.0)
    xn = _ln(h, gn[...][None, :], ben[...][None, :])
    xo[...] = xn
    to[...] = _mm(xn, W1pq[...])


def _node_upd_split_body(*refs):
    ags = refs[:SEG]
    (x, Wna, Wnb, b1n, W2n, b2n, gn, ben, W1p, W1q, xo, tpo, tqo) = refs[SEG:]
    a = sum(ag[...][0] + ag[...][1] for ag in ags)
    pre = _mm(a, Wna[...]) + _mm(x[...], Wnb[...]) + b1n[...][None, :]
    h = jnp.maximum(_mm(jnp.maximum(pre, 0.0), W2n[...]) + b2n[...][None, :], 0.0)
    xn = _ln(h, gn[...][None, :], ben[...][None, :])
    xo[...] = xn
    tpo[...] = _mm(xn, W1p[...])
    tqo[...] = _mm(xn, W1q[...])


def _node_fin_body(*refs):
    ags = refs[:SEG]
    (x, Wna, Wnb, b1n, W2n, b2n, gn, ben, Wf1, bf1, Wf2, bf2, oo) = refs[SEG:]
    a = sum(ag[...][0] + ag[...][1] for ag in ags)
    pre = _mm(a, Wna[...]) + _mm(x[...], Wnb[...]) + b1n[...][None, :]
    h = jnp.maximum(_mm(jnp.maximum(pre, 0.0), W2n[...]) + b2n[...][None, :], 0.0)
    xn = _ln(h, gn[...][None, :], ben[...][None, :])
    h2 = jnp.maximum(_mm(xn, Wf1[...]) + bf1[...][None, :], 0.0)
    oo[...] = _mm(h2, Wf2[...]) + bf2[...][None, :]


def _row_spec(blk, width):
    return pl.BlockSpec((blk, width), lambda i: (i, 0))


def _seg_spec(blk, width, seg):
    off = seg * SBLKS
    return pl.BlockSpec((blk, width), lambda i, _o=off: (_o + i, 0))


_AG_SPEC = pl.BlockSpec((2, NBLK, H), lambda i: (0, i, 0))
_AG_SHAPE = jax.ShapeDtypeStruct((NCORE, NPAD, H), _f32)


def _node0_call(x, W1, b1, W2, b2, g, be, W1pq):
    nb = N // NBLK
    wspecs = [_full(w.shape) for w in (W1, b1, W2, b2, g, be, W1pq)]
    return pl.pallas_call(
        _node0_body,
        grid=(nb,),
        in_specs=[_row_spec(NBLK, H)] + wspecs,
        out_specs=[_row_spec(NBLK, H), _row_spec(NBLK, H2)],
        out_shape=[jax.ShapeDtypeStruct((N, H), _f32),
                   jax.ShapeDtypeStruct((N, H2), _f32)],
    )(x, W1, b1, W2, b2, g, be, W1pq)


def _edge0_call(ea, seg, W1, b1, W2, b2, g, be, W1c, b1e):
    ws = (W1, b1, W2, b2, g, be, W1c, b1e)
    wspecs = [_full(w.shape) for w in ws]
    return pl.pallas_call(
        _edge0_body,
        grid=(SBLKS,),
        in_specs=[_seg_spec(EBLK, ea.shape[1], seg)] + wspecs,
        out_specs=_row_spec(EBLK, H),
        out_shape=jax.ShapeDtypeStruct((ESEG, H), _f32),
    )(ea, *ws)


def _edge_main_call(p1, p2, c, W2, b2, g, be, W1c, b1e):
    ws = (W2, b2, g, be, W1c, b1e)
    wspecs = [_full(w.shape) for w in ws]
    return pl.pallas_call(
        _edge_main_body,
        grid=(SBLKS,),
        in_specs=[_row_spec(EBLK, H)] * 3 + wspecs,
        out_specs=[_row_spec(EBLK, H)] * 2,
        out_shape=[jax.ShapeDtypeStruct((ESEG, H), _f32)] * 2,
    )(p1, p2, c, *ws)


def _edge_last_call(p1, c, W2, b2, g, be):
    ws = (W2, b2, g, be)
    wspecs = [_full(w.shape) for w in ws]
    return pl.pallas_call(
        _edge_last_body,
        grid=(SBLKS,),
        in_specs=[_row_spec(EBLK, H)] * 2 + wspecs,
        out_specs=_row_spec(EBLK, H),
        out_shape=jax.ShapeDtypeStruct((ESEG, H), _f32),
    )(p1, c, *ws)


def _node_upd_call(ags, x, Wna, Wnb, b1n, W2n, b2n, gn, ben, W1pq):
    nb = N // NBLK
    ws = (Wna, Wnb, b1n, W2n, b2n, gn, ben, W1pq)
    wspecs = [_full(w.shape) for w in ws]
    return pl.pallas_call(
        _node_upd_body,
        grid=(nb,),
        in_specs=[_AG_SPEC] * SEG + [_row_spec(NBLK, H)] + wspecs,
        out_specs=[_row_spec(NBLK, H), _row_spec(NBLK, H2)],
        out_shape=[jax.ShapeDtypeStruct((N, H), _f32),
                   jax.ShapeDtypeStruct((N, H2), _f32)],
    )(*ags, x, *ws)


def _node_upd_split_call(ags, x, Wna, Wnb, b1n, W2n, b2n, gn, ben, W1p, W1q):
    nb = N // NBLK
    ws = (Wna, Wnb, b1n, W2n, b2n, gn, ben, W1p, W1q)
    wspecs = [_full(w.shape) for w in ws]
    return pl.pallas_call(
        _node_upd_split_body,
        grid=(nb,),
        in_specs=[_AG_SPEC] * SEG + [_row_spec(NBLK, H)] + wspecs,
        out_specs=[_row_spec(NBLK, H)] * 3,
        out_shape=[jax.ShapeDtypeStruct((N, H), _f32)] * 3,
    )(*ags, x, *ws)


def _node_fin_call(ags, x, Wna, Wnb, b1n, W2n, b2n, gn, ben, Wf1, bf1, Wf2, bf2):
    nb = N // NBLK
    ws = (Wna, Wnb, b1n, W2n, b2n, gn, ben, Wf1, bf1, Wf2, bf2)
    wspecs = [_full(w.shape) for w in ws]
    dout = Wf2.shape[1]
    return pl.pallas_call(
        _node_fin_body,
        grid=(nb,),
        in_specs=[_AG_SPEC] * SEG + [_row_spec(NBLK, H)] + wspecs,
        out_specs=_row_spec(NBLK, dout),
        out_shape=jax.ShapeDtypeStruct((N, dout), _f32),
    )(*ags, x, *ws)


_MESH = plsc.VectorSubcoreMesh(core_axis_name="c", subcore_axis_name="s")


def _wid():
    return lax.axis_index("s") * NCORE + lax.axis_index("c")


def _make_gather_both(seg):
    seg_base = seg * ESEG
    out_type = (jax.ShapeDtypeStruct((ESEG, H), _f32),
                jax.ShapeDtypeStruct((ESEG, H), _f32))
    scratch = (
        [pltpu.VMEM((CHUNK,), _i32)] * 6 +
        [pltpu.VMEM((CHUNK, H2), _f32)] * 6 +
        [pltpu.SemaphoreType.DMA] * 9
    )

    def body(th, colh, rowh, dummy, o1h, o2h,
             cb0, cb1, cb2, rb0, rb1, rb2, gc0, gc1, gc2, gr0, gr1, gr2,
             si0, si1, si2, sg0, sg1, sg2, sw0, sw1, sw2):
        del dummy
        colb = (cb0, cb1, cb2)
        rowb = (rb0, rb1, rb2)
        gc = (gc0, gc1, gc2)
        gr = (gr0, gr1, gr2)
        si = (si0, si1, si2)
        sg = (sg0, sg1, sg2)
        sw = (sw0, sw1, sw2)
        obase0 = _wid() * EPW
        base0 = seg_base + obase0

        def idx_start(g, b):
            base = base0 + g * CHUNK
            pltpu.async_copy(colh.at[pl.ds(base, CHUNK)], colb[b], si[b])
            pltpu.async_copy(rowh.at[pl.ds(base, CHUNK)], rowb[b], si[b])

        def idx_wait(b):
            pltpu.make_async_copy(colh.at[pl.ds(0, CHUNK)], colb[b], si[b]).wait()
            pltpu.make_async_copy(rowh.at[pl.ds(0, CHUNK)], rowb[b], si[b]).wait()

        def g_start(b):
            pltpu.async_copy(th.at[colb[b]], gc[b], sg[b])
            pltpu.async_copy(th.at[rowb[b]], gr[b], sg[b])

        def g_wait(b):
            pltpu.make_async_copy(th.at[colb[b]], gc[b], sg[b]).wait()
            pltpu.make_async_copy(th.at[rowb[b]], gr[b], sg[b]).wait()

        def w_start(g, b):
            base = obase0 + g * CHUNK
            pltpu.async_copy(gc[b].at[:, pl.ds(0, H)], o1h.at[pl.ds(base, CHUNK)], sw[b])
            pltpu.async_copy(gr[b].at[:, pl.ds(0, H)], o2h.at[pl.ds(base, CHUNK)], sw[b])

        def w_wait(b):
            pltpu.make_async_copy(gc[b].at[:, pl.ds(0, H)], o1h.at[pl.ds(0, CHUNK)], sw[b]).wait()
            pltpu.make_async_copy(gr[b].at[:, pl.ds(0, H)], o2h.at[pl.ds(0, CHUNK)], sw[b]).wait()

        def combine(b):
            gcb, grb = gc[b], gr[b]

            @plsc.parallel_loop(0, CHUNK, 1, unroll=4)
            def _(i):
                for j in range(H // 16):
                    sl = pl.ds(j * 16, 16)
                    sh = pl.ds(H + j * 16, 16)
                    gcb[i, sl] = gcb[i, sl] + grb[i, sh]
                    grb[i, sl] = grb[i, sl] + gcb[i, sh]

        idx_start(0, 0)
        idx_wait(0)
        g_start(0)
        idx_start(1, 1)
        idx_wait(1)
        g_start(1)
        idx_start(2, 2)

        def triple(t, _):
            for b in range(3):
                g = 3 * t + b
                c = (b + 2) % 3

                @pl.when(g < NCHUNK)
                def _():
                    g_wait(b)

                    @pl.when(g + 2 < NCHUNK)
                    def _():
                        idx_wait(c)

                        @pl.when(g >= 1)
                        def _():
                            w_wait(c)

                        g_start(c)

                    @pl.when(g + 3 < NCHUNK)
                    def _():
                        idx_start(g + 3, b)

                    combine(b)
                    w_start(g, b)
            return 0

        lax.fori_loop(0, (NCHUNK + 2) // 3, triple, 0)
        for gg in (NCHUNK - 3, NCHUNK - 2, NCHUNK - 1):
            if gg >= 0:
                w_wait(gg % 3)

    return pl.kernel(body, out_type=out_type, mesh=_MESH, scratch_types=scratch)


def _make_gather_one(seg):
    seg_base = seg * ESEG
    out_type = jax.ShapeDtypeStruct((ESEG, H), _f32)
    scratch = (
        [pltpu.VMEM((CHUNK,), _i32)] * 6 +
        [pltpu.VMEM((CHUNK, H), _f32)] * 6 +
        [pltpu.SemaphoreType.DMA] * 9
    )

    def body(tph, tqh, colh, rowh, dummy, o1h,
             cb0, cb1, cb2, rb0, rb1, rb2, gp0, gp1, gp2, gq0, gq1, gq2,
             si0, si1, si2, sg0, sg1, sg2, sw0, sw1, sw2):
        del dummy
        colb = (cb0, cb1, cb2)
        rowb = (rb0, rb1, rb2)
        gp = (gp0, gp1, gp2)
        gq = (gq0, gq1, gq2)
        si = (si0, si1, si2)
        sg = (sg0, sg1, sg2)
        sw = (sw0, sw1, sw2)
        obase0 = _wid() * EPW
        base0 = seg_base + obase0

        def idx_start(g, b):
            base = base0 + g * CHUNK
            pltpu.async_copy(colh.at[pl.ds(base, CHUNK)], colb[b], si[b])
            pltpu.async_copy(rowh.at[pl.ds(base, CHUNK)], rowb[b], si[b])

        def idx_wait(b):
            pltpu.make_async_copy(colh.at[pl.ds(0, CHUNK)], colb[b], si[b]).wait()
            pltpu.make_async_copy(rowh.at[pl.ds(0, CHUNK)], rowb[b], si[b]).wait()

        def g_start(b):
            pltpu.async_copy(tph.at[colb[b]], gp[b], sg[b])
            pltpu.async_copy(tqh.at[rowb[b]], gq[b], sg[b])

        def g_wait(b):
            pltpu.make_async_copy(tph.at[colb[b]], gp[b], sg[b]).wait()
            pltpu.make_async_copy(tqh.at[rowb[b]], gq[b], sg[b]).wait()

        def w_start(g, b):
            base = obase0 + g * CHUNK
            pltpu.async_copy(gp[b], o1h.at[pl.ds(base, CHUNK)], sw[b])

        def w_wait(b):
            pltpu.make_async_copy(gp[b], o1h.at[pl.ds(0, CHUNK)], sw[b]).wait()

        def combine(b):
            gpb, gqb = gp[b], gq[b]

            @plsc.parallel_loop(0, CHUNK, 1, unroll=4)
            def _(i):
                for j in range(H // 16):
                    sl = pl.ds(j * 16, 16)
                    gpb[i, sl] = gpb[i, sl] + gqb[i, sl]

        idx_start(0, 0)
        idx_wait(0)
        g_start(0)
        idx_start(1, 1)
        idx_wait(1)
        g_start(1)
        idx_start(2, 2)

        def triple(t, _):
            for b in range(3):
                g = 3 * t + b
                c = (b + 2) % 3

                @pl.when(g < NCHUNK)
                def _():
                    g_wait(b)

                    @pl.when(g + 2 < NCHUNK)
                    def _():
                        idx_wait(c)

                        @pl.when(g >= 1)
                        def _():
                            w_wait(c)

                        g_start(c)

                    @pl.when(g + 3 < NCHUNK)
                    def _():
                        idx_start(g + 3, b)

                    combine(b)
                    w_start(g, b)
            return 0

        lax.fori_loop(0, (NCHUNK + 2) // 3, triple, 0)
        for gg in (NCHUNK - 3, NCHUNK - 2, NCHUNK - 1):
            if gg >= 0:
                w_wait(gg % 3)

    return pl.kernel(body, out_type=out_type, mesh=_MESH, scratch_types=scratch)


def _make_scatter_multi(seg_list):
  nseg = len(seg_list)

  def _scatter_body(*refs):
    msgs = refs[:nseg]
    (colh, dummy, outh, colb0, colb1, mb0, mb1, zb, shared,
     si0, si1, sc0, sc1) = refs[nseg:]
    del dummy
    cid = lax.axis_index("c")
    sid = lax.axis_index("s")
    wid = sid * NCORE + cid
    colb = (colb0, colb1)
    mb = (mb0, mb1)
    si = (si0, si1)
    sc = (sc0, sc1)
    zero = jnp.zeros((16,), _f32)

    def zrow(i, _):
        for j in range(H // 16):
            zb[i, pl.ds(j * 16, 16)] = zero
        return 0

    lax.fori_loop(0, ZROWS, zrow, 0)
    for t in range(ROWS_PER_SUB // ZROWS):
        pltpu.sync_copy(zb, shared.at[pl.ds(sid * ROWS_PER_SUB + t * ZROWS, ZROWS)])
    plsc.subcore_barrier()

    base0 = wid * EPW
    for si_, s in enumerate(seg_list):
        msgh = msgs[si_]
        seg_base = s * ESEG

        def in_start(g, b):
            base = base0 + g * CHUNK
            pltpu.async_copy(colh.at[pl.ds(seg_base + base, CHUNK)], colb[b], si[b])
            pltpu.async_copy(msgh.at[pl.ds(base, CHUNK)], mb[b], si[b])

        def in_wait(b):
            pltpu.make_async_copy(colh.at[pl.ds(0, CHUNK)], colb[b], si[b]).wait()
            pltpu.make_async_copy(msgh.at[pl.ds(0, CHUNK)], mb[b], si[b]).wait()

        def sc_start(b):
            pltpu.async_copy(mb[b], shared.at[colb[b]], sc[b], add=True)

        def sc_wait(b):
            pltpu.make_async_copy(mb[b], shared.at[colb[b]], sc[b]).wait()

        in_start(0, 0)

        def pair(t, _):
            for b in range(2):
                g = 2 * t + b
                nb_ = 1 - b

                @pl.when(g < NCHUNK)
                def _():
                    in_wait(b)
                    sc_start(b)

                    @pl.when(g >= 1)
                    def _():
                        sc_wait(nb_)

                    @pl.when(g + 1 < NCHUNK)
                    def _():
                        in_start(g + 1, nb_)
            return 0

        lax.fori_loop(0, NPAIR, pair, 0)
        sc_wait((NCHUNK - 1) % 2)
    plsc.subcore_barrier()
    for t in range(ROWS_PER_SUB // ZROWS):
        r0 = sid * ROWS_PER_SUB + t * ZROWS
        pltpu.sync_copy(shared.at[pl.ds(r0, ZROWS)], zb)
        pltpu.sync_copy(zb, outh.at[cid, pl.ds(r0, ZROWS)])


  return pl.kernel(
      _scatter_body,
      out_type=_AG_SHAPE,
      mesh=_MESH,
      scratch_types=[
          pltpu.VMEM((CHUNK,), _i32),
          pltpu.VMEM((CHUNK,), _i32),
          pltpu.VMEM((CHUNK, H), _f32),
          pltpu.VMEM((CHUNK, H), _f32),
          pltpu.VMEM((ZROWS, H), _f32),
          pltpu.VMEM_SHARED((NPAD, H), _f32),
          pltpu.SemaphoreType.DMA,
          pltpu.SemaphoreType.DMA,
          pltpu.SemaphoreType.DMA,
          pltpu.SemaphoreType.DMA,
      ],
  )


_scatter_k = [_make_scatter_multi([s]) for s in range(SEG)]


_gather_both_k = [_make_gather_both(s) for s in range(SEG)]
_gather_one_k = [_make_gather_one(s) for s in range(SEG)]


def _tok(a):
    if a.ndim == 3:
        return a[0, 0, :8]
    return a[0, :8]


def kernel(x, edge_index, edge_attr, params):
    row = edge_index[0].astype(_i32)
    col = edge_index[1].astype(_i32)
    n0 = params['nodeMLP0']
    e0 = params['edgeMLP0']
    ge = params['gnn_edge']
    gn = params['gnn_node']
    fin = params['final']
    W1p = ge['W1'][:H]
    W1q = ge['W1'][H:2 * H]
    W1pq = jnp.concatenate([W1p, W1q], axis=1)
    W1c = ge['W1'][2 * H:]
    Wna = gn['W1'][:H]
    Wnb = gn['W1'][H:]

    x_, tpq = _node0_call(x, n0['W1'], n0['b1'], n0['W2'], n0['b2'],
                          n0['g'], n0['be'], W1pq)
    cs = [_edge0_call(edge_attr, s, e0['W1'], e0['b1'], e0['W2'], e0['b2'],
                      e0['g'], e0['be'], W1c, ge['b1'])
          for s in range(SEG)]

    tp = tq = None
    out = None
    tok = jnp.zeros((8,), _f32)
    for r in range(3):
        pres = [None] * SEG
        ncs = [None] * SEG

        def do_gather(s, tk):
            if r < 2:
                p = _gather_both_k[s](tpq, col, row, tk)
                return p, _tok(p[0])
            p = _gather_one_k[s](tp, tq, col, row, tk)
            return p, _tok(p)

        def do_edge(s):
            if r < 2:
                msg, cn = _edge_main_call(pres[s][0], pres[s][1], cs[s],
                                          ge['W2'], ge['b2'], ge['g'], ge['be'],
                                          W1c, ge['b1'])
                ncs[s] = cn
                return msg
            return _edge_last_call(pres[s], cs[s], ge['W2'], ge['b2'],
                                   ge['g'], ge['be'])

        ags = [None] * SEG
        pres[0], tok = do_gather(0, tok)
        for s in range(1, SEG + 1):
            if s < SEG:
                pres[s], tok = do_gather(s, tok)
            msg = do_edge(s - 1)
            ags[s - 1] = _scatter_k[s - 1](msg, col, tok)
            tok = _tok(ags[s - 1])
        if r < 2:
            cs = ncs
        if r == 0:
            x_, tpq = _node_upd_call(ags, x_, Wna, Wnb, gn['b1'],
                                     gn['W2'], gn['b2'], gn['g'], gn['be'], W1pq)
        elif r == 1:
            x_, tp, tq = _node_upd_split_call(ags, x_, Wna, Wnb, gn['b1'],
                                              gn['W2'], gn['b2'], gn['g'],
                                              gn['be'], W1p, W1q)
        else:
            out = _node_fin_call(ags, x_, Wna, Wnb, gn['b1'],
                                 gn['W2'], gn['b2'], gn['g'], gn['be'],
                                 fin['W1'], fin['b1'], fin['W2'], fin['b2'])
    return out
